# baseline (speedup 1.0000x reference)
import functools

import jax
import jax.numpy as jnp
from jax import lax
from jax.experimental import pallas as pl
from jax.experimental.pallas import tpu as pltpu
from jax.experimental.pallas import tpu_sc as plsc

B = 16384
NC, NS = 2, 16
NW = NC * NS
BPW = B // NW
CH = 128
NCHUNK = BPW // CH
PH = 2
P = BPW // PH

TBLK = 8192


def _transpose_body(xT_ref, eye_ref, out_ref):
    out_ref[...] = lax.dot_general(
        xT_ref[...], eye_ref[...], (((0,), (0,)), ((), ())),
        preferred_element_type=jnp.float32)


def _to_row_major(xT):
    F, N = xT.shape
    eye = jnp.eye(F, dtype=jnp.float32)
    grid = (pl.cdiv(N, TBLK),)
    return pl.pallas_call(
        _transpose_body,
        grid=grid,
        in_specs=[
            pl.BlockSpec((F, TBLK), lambda i: (0, i)),
            pl.BlockSpec((F, F), lambda i: (0, 0)),
        ],
        out_specs=pl.BlockSpec((TBLK, F), lambda i: (i, 0)),
        out_shape=jax.ShapeDtypeStruct((N, F), jnp.float32),
    )(xT, eye)


def _transpose_pack4_body(xT_ref, eye_ref, out_ref):
    Q = TBLK // 4
    for q in range(4):
        y = lax.dot_general(
            xT_ref[:, q * Q:(q + 1) * Q], eye_ref[...],
            (((0,), (0,)), ((), ())), preferred_element_type=jnp.float32)
        out_ref[:, 32 * q:32 * (q + 1)] = y


def _to_row_major_pack4(xT):
    F, N = xT.shape
    eye = jnp.eye(F, dtype=jnp.float32)
    nblk = pl.cdiv(N, TBLK)
    return pl.pallas_call(
        _transpose_pack4_body,
        grid=(nblk,),
        in_specs=[
            pl.BlockSpec((F, TBLK), lambda i: (0, i)),
            pl.BlockSpec((F, F), lambda i: (0, 0)),
        ],
        out_specs=pl.BlockSpec((TBLK // 4, 128), lambda i: (i, 0)),
        out_shape=jax.ShapeDtypeStruct((nblk * (TBLK // 4), 128), jnp.float32),
    )(xT, eye)


@functools.lru_cache(maxsize=None)
def _make_sc_gather():
    mesh = plsc.VectorSubcoreMesh(
        core_axis_name="c", subcore_axis_name="s",
        num_cores=NC, num_subcores=NS)

    @functools.partial(
        pl.kernel,
        out_type=(
            jax.ShapeDtypeStruct((B, 128), jnp.float32),
            jax.ShapeDtypeStruct((B, 32), jnp.float32),
            jax.ShapeDtypeStruct((B, 8), jnp.float32),
        ),
        mesh=mesh,
        scratch_types=(
            pltpu.VMEM((NCHUNK, CH), jnp.int32),
            pltpu.VMEM((NCHUNK, CH), jnp.int32),
            pltpu.VMEM((NCHUNK, CH), jnp.int32),
            pltpu.VMEM((P, 128), jnp.float32),
            pltpu.VMEM((P, 32), jnp.float32),
            pltpu.VMEM((P, 8), jnp.float32),
            pltpu.SemaphoreType.DMA,
        ),
    )
    def _sc_gather(user_hbm, movie_hbm, region_hbm, ut_hbm, mt_hbm, rt_hbm,
                   u_out, m_out, r_out,
                   uidx, midx, ridx, urows, mrows, rrows, sem):
        wid = lax.axis_index("s") * NC + lax.axis_index("c")
        row0 = wid * NCHUNK

        pltpu.sync_copy(user_hbm.at[pl.ds(row0, NCHUNK)], uidx)
        pltpu.sync_copy(movie_hbm.at[pl.ds(row0, NCHUNK)], midx)
        pltpu.sync_copy(region_hbm.at[pl.ds(row0, NCHUNK)], ridx)

        base = wid * BPW
        for h in range(PH):
            goff = h * (P // 16)

            def body(g, carry):
                i16 = g * 16
                j = (goff * 16 + i16) // CH
                k = (goff * 16 + i16) % CH
                uvec = uidx[j, pl.ds(k, 16)]
                mvec = midx[j, pl.ds(k, 16)]
                rvec = ridx[j, pl.ds(k, 16)]
                for l in range(16):
                    pltpu.async_copy(ut_hbm.at[uvec[l]], urows.at[i16 + l], sem)
                    pltpu.async_copy(mt_hbm.at[mvec[l]], mrows.at[i16 + l], sem)
                    pltpu.async_copy(rt_hbm.at[rvec[l]], rrows.at[i16 + l], sem)
                return carry

            lax.fori_loop(0, P // 16, body, 0)

            off = base + h * P
            pltpu.make_async_copy(u_out.at[pl.ds(off, P)], urows, sem).wait()
            pltpu.make_async_copy(m_out.at[pl.ds(off, P)], mrows, sem).wait()
            pltpu.make_async_copy(r_out.at[pl.ds(off, P)], rrows, sem).wait()

            pltpu.sync_copy(urows, u_out.at[pl.ds(off, P)])
            pltpu.sync_copy(mrows, m_out.at[pl.ds(off, P)])
            pltpu.sync_copy(rrows, r_out.at[pl.ds(off, P)])

    return _sc_gather


BLK = 2048


def _select32(w128, pos):
    out = w128[:, 0:32]
    for p in (1, 2, 3):
        out = jnp.where(pos == p, w128[:, 32 * p:32 * p + 32], out)
    return out


def _dense_body(u_ref, m_ref, r_ref, kw_ref, age_ref, upos_ref,
                kwW_ref, kwb_ref, W1u_ref, W1m_ref, W1r_ref, W1k_ref,
                w1a_ref, b1_ref, W2_ref, b2_ref, out_ref):
    u = _select32(u_ref[...], upos_ref[...])
    k = jnp.maximum(
        jnp.dot(kw_ref[...], kwW_ref[...], preferred_element_type=jnp.float32)
        + kwb_ref[...], 0.0)
    acc = jnp.dot(u, W1u_ref[...], preferred_element_type=jnp.float32)
    acc += jnp.dot(m_ref[...], W1m_ref[...], preferred_element_type=jnp.float32)
    acc += jnp.dot(r_ref[...], W1r_ref[...], preferred_element_type=jnp.float32)
    acc += jnp.dot(k, W1k_ref[...], preferred_element_type=jnp.float32)
    acc += age_ref[...] * w1a_ref[...]
    h = jnp.maximum(acc + b1_ref[...], 0.0)
    o = jnp.dot(h, W2_ref[...], preferred_element_type=jnp.float32) + b2_ref[...]
    out_ref[...] = 1.0 / (1.0 + jnp.exp(-o))


def _dense(u, m, r, keywords, age2d, upos, kwW, kwb, W1u, W1m, W1r, W1k,
           w1a, b1, W2, b2):
    grid = (B // BLK,)
    blk = lambda w: pl.BlockSpec((BLK, w), lambda i: (i, 0))
    rep = lambda s0, s1: pl.BlockSpec((s0, s1), lambda i: (0, 0))
    return pl.pallas_call(
        _dense_body,
        grid=grid,
        in_specs=[
            blk(128), blk(32), blk(8), blk(64), blk(1), blk(1),
            rep(64, 16), rep(1, 16), rep(32, 64), rep(32, 64), rep(8, 64),
            rep(16, 64), rep(1, 64), rep(1, 64), rep(64, 1), rep(1, 1),
        ],
        out_specs=blk(1),
        out_shape=jax.ShapeDtypeStruct((B, 1), jnp.float32),
    )(u, m, r, keywords, age2d, upos, kwW, kwb, W1u, W1m, W1r, W1k, w1a, b1,
      W2, b2)


def kernel(user, movie, region, keywords, age, user_table, movie_table,
           region_table, kw_W, kw_b, W1, b1, W2, b2):
    ut_p4 = _to_row_major_pack4(user_table.T)
    mt_rm = _to_row_major(movie_table.T)
    rt_rm = _to_row_major(region_table.T)
    Q = TBLK // 4
    urow = (user >> 13) * (TBLK // 4) + (user & (Q - 1))
    upos = (user >> 11) & 3
    u, m, r = _make_sc_gather()(
        urow.reshape(NW * NCHUNK, CH),
        movie.reshape(NW * NCHUNK, CH),
        region.reshape(NW * NCHUNK, CH),
        ut_p4, mt_rm, rt_rm)
    out = _dense(
        u, m, r, keywords, age[:, None], upos[:, None],
        kw_W, kw_b[None, :],
        W1[0:32], W1[32:64], W1[64:72], W1[72:88], W1[88:89],
        b1[None, :], W2, b2[None, :])
    return out[:, 0]


# speedup vs baseline: 1.3539x; 1.3539x over previous
import functools

import jax
import jax.numpy as jnp
from jax import lax
from jax.experimental import pallas as pl
from jax.experimental.pallas import tpu as pltpu
from jax.experimental.pallas import tpu_sc as plsc

B = 16384
NC, NS = 2, 16
NW = NC * NS
BPW = B // NW
CH = 128
NCHUNK = BPW // CH
PH = 2
P = BPW // PH

TBLK = 32768


def _transpose_body(xT_ref, eye_ref, out_ref):
    out_ref[...] = lax.dot_general(
        xT_ref[...], eye_ref[...], (((0,), (0,)), ((), ())),
        preferred_element_type=jnp.float32)


def _to_row_major(xT):
    F, N = xT.shape
    eye = jnp.eye(F, dtype=jnp.float32)
    grid = (pl.cdiv(N, TBLK),)
    return pl.pallas_call(
        _transpose_body,
        grid=grid,
        in_specs=[
            pl.BlockSpec((F, TBLK), lambda i: (0, i)),
            pl.BlockSpec((F, F), lambda i: (0, 0)),
        ],
        out_specs=pl.BlockSpec((TBLK, F), lambda i: (i, 0)),
        out_shape=jax.ShapeDtypeStruct((N, F), jnp.float32),
    )(xT, eye)


def _transpose_pack4_body(xT_ref, sel_ref, out_ref):
    Q = TBLK // 4
    acc = None
    for q in range(4):
        y = lax.dot_general(
            xT_ref[:, q * Q:(q + 1) * Q], sel_ref[:, 128 * q:128 * (q + 1)],
            (((0,), (0,)), ((), ())), preferred_element_type=jnp.float32)
        acc = y if acc is None else acc + y
    out_ref[...] = acc


def _to_row_major_pack4(xT):
    F, N = xT.shape
    rows = jnp.tile(jnp.arange(32), 4)
    cols = (jnp.repeat(jnp.arange(4) * 160, 32) + jnp.tile(jnp.arange(32), 4))
    sel = jnp.zeros((32, 512), jnp.float32).at[rows, cols].set(1.0)
    nblk = pl.cdiv(N, TBLK)
    return pl.pallas_call(
        _transpose_pack4_body,
        grid=(nblk,),
        in_specs=[
            pl.BlockSpec((F, TBLK), lambda i: (0, i)),
            pl.BlockSpec((32, 512), lambda i: (0, 0)),
        ],
        out_specs=pl.BlockSpec((TBLK // 4, 128), lambda i: (i, 0)),
        out_shape=jax.ShapeDtypeStruct((nblk * (TBLK // 4), 128), jnp.float32),
    )(xT, sel)


@functools.lru_cache(maxsize=None)
def _make_sc_gather():
    mesh = plsc.VectorSubcoreMesh(
        core_axis_name="c", subcore_axis_name="s",
        num_cores=NC, num_subcores=NS)

    @functools.partial(
        pl.kernel,
        out_type=(
            jax.ShapeDtypeStruct((B, 128), jnp.float32),
            jax.ShapeDtypeStruct((B, 32), jnp.float32),
            jax.ShapeDtypeStruct((B, 8), jnp.float32),
        ),
        mesh=mesh,
        scratch_types=(
            pltpu.VMEM((NCHUNK, CH), jnp.int32),
            pltpu.VMEM((NCHUNK, CH), jnp.int32),
            pltpu.VMEM((NCHUNK, CH), jnp.int32),
            pltpu.VMEM((P, 128), jnp.float32),
            pltpu.VMEM((P, 32), jnp.float32),
            pltpu.VMEM((P, 8), jnp.float32),
            pltpu.SemaphoreType.DMA,
        ),
    )
    def _sc_gather(user_hbm, movie_hbm, region_hbm, ut_hbm, mt_hbm, rt_hbm,
                   u_out, m_out, r_out,
                   uidx, midx, ridx, urows, mrows, rrows, sem):
        wid = lax.axis_index("s") * NC + lax.axis_index("c")
        row0 = wid * NCHUNK

        pltpu.sync_copy(user_hbm.at[pl.ds(row0, NCHUNK)], uidx)
        pltpu.sync_copy(movie_hbm.at[pl.ds(row0, NCHUNK)], midx)
        pltpu.sync_copy(region_hbm.at[pl.ds(row0, NCHUNK)], ridx)

        base = wid * BPW
        for h in range(PH):
            goff = h * (P // 16)

            def body(g, carry):
                i16 = g * 16
                j = (goff * 16 + i16) // CH
                k = (goff * 16 + i16) % CH
                uvec = uidx[j, pl.ds(k, 16)]
                mvec = midx[j, pl.ds(k, 16)]
                rvec = ridx[j, pl.ds(k, 16)]
                for l in range(16):
                    pltpu.async_copy(ut_hbm.at[uvec[l]], urows.at[i16 + l], sem)
                    pltpu.async_copy(mt_hbm.at[mvec[l]], mrows.at[i16 + l], sem)
                    pltpu.async_copy(rt_hbm.at[rvec[l]], rrows.at[i16 + l], sem)
                return carry

            lax.fori_loop(0, P // 16, body, 0)

            off = base + h * P
            pltpu.make_async_copy(u_out.at[pl.ds(off, P)], urows, sem).wait()
            pltpu.make_async_copy(m_out.at[pl.ds(off, P)], mrows, sem).wait()
            pltpu.make_async_copy(r_out.at[pl.ds(off, P)], rrows, sem).wait()

            pltpu.sync_copy(urows, u_out.at[pl.ds(off, P)])
            pltpu.sync_copy(mrows, m_out.at[pl.ds(off, P)])
            pltpu.sync_copy(rrows, r_out.at[pl.ds(off, P)])

    return _sc_gather


BLK = 2048


def _select32(w128, pos):
    out = w128[:, 0:32]
    for p in (1, 2, 3):
        out = jnp.where(pos == p, w128[:, 32 * p:32 * p + 32], out)
    return out


def _dense_body(u_ref, m_ref, r_ref, kw_ref, age_ref, upos_ref,
                kwW_ref, kwb_ref, W1u_ref, W1m_ref, W1r_ref, W1k_ref,
                w1a_ref, b1_ref, W2_ref, b2_ref, out_ref):
    u = _select32(u_ref[...], upos_ref[...])
    k = jnp.maximum(
        jnp.dot(kw_ref[...], kwW_ref[...], preferred_element_type=jnp.float32)
        + kwb_ref[...], 0.0)
    acc = jnp.dot(u, W1u_ref[...], preferred_element_type=jnp.float32)
    acc += jnp.dot(m_ref[...], W1m_ref[...], preferred_element_type=jnp.float32)
    acc += jnp.dot(r_ref[...], W1r_ref[...], preferred_element_type=jnp.float32)
    acc += jnp.dot(k, W1k_ref[...], preferred_element_type=jnp.float32)
    acc += age_ref[...] * w1a_ref[...]
    h = jnp.maximum(acc + b1_ref[...], 0.0)
    o = jnp.dot(h, W2_ref[...], preferred_element_type=jnp.float32) + b2_ref[...]
    out_ref[...] = 1.0 / (1.0 + jnp.exp(-o))


def _dense(u, m, r, keywords, age2d, upos, kwW, kwb, W1u, W1m, W1r, W1k,
           w1a, b1, W2, b2):
    grid = (B // BLK,)
    blk = lambda w: pl.BlockSpec((BLK, w), lambda i: (i, 0))
    rep = lambda s0, s1: pl.BlockSpec((s0, s1), lambda i: (0, 0))
    return pl.pallas_call(
        _dense_body,
        grid=grid,
        in_specs=[
            blk(128), blk(32), blk(8), blk(64), blk(1), blk(1),
            rep(64, 16), rep(1, 16), rep(32, 64), rep(32, 64), rep(8, 64),
            rep(16, 64), rep(1, 64), rep(1, 64), rep(64, 1), rep(1, 1),
        ],
        out_specs=blk(1),
        out_shape=jax.ShapeDtypeStruct((B, 1), jnp.float32),
    )(u, m, r, keywords, age2d, upos, kwW, kwb, W1u, W1m, W1r, W1k, w1a, b1,
      W2, b2)


def kernel(user, movie, region, keywords, age, user_table, movie_table,
           region_table, kw_W, kw_b, W1, b1, W2, b2):
    ut_p4 = _to_row_major_pack4(user_table.T)
    mt_rm = _to_row_major(movie_table.T)
    rt_rm = _to_row_major(region_table.T)
    Q = TBLK // 4
    urow = (user // TBLK) * Q + (user % Q)
    upos = (user // Q) % 4
    u, m, r = _make_sc_gather()(
        urow.reshape(NW * NCHUNK, CH),
        movie.reshape(NW * NCHUNK, CH),
        region.reshape(NW * NCHUNK, CH),
        ut_p4, mt_rm, rt_rm)
    out = _dense(
        u, m, r, keywords, age[:, None], upos[:, None],
        kw_W, kw_b[None, :],
        W1[0:32], W1[32:64], W1[64:72], W1[72:88], W1[88:89],
        b1[None, :], W2, b2[None, :])
    return out[:, 0]


# speedup vs baseline: 1.4025x; 1.0358x over previous
import functools

import jax
import jax.numpy as jnp
import numpy as np
from jax import lax
from jax.experimental import pallas as pl
from jax.experimental.pallas import tpu as pltpu
from jax.experimental.pallas import tpu_sc as plsc

B = 16384
NC, NS = 2, 16
NW = NC * NS
BPW = B // NW
CH = 128
NCHUNK = BPW // CH
PH = 2
P = BPW // PH

TBLK = 32768


def _transpose_body(xT_ref, eye_ref, out_ref):
    out_ref[...] = lax.dot_general(
        xT_ref[...], eye_ref[...], (((0,), (0,)), ((), ())),
        preferred_element_type=jnp.float32)


def _to_row_major(xT):
    F, N = xT.shape
    eye = jnp.eye(F, dtype=jnp.float32)
    grid = (pl.cdiv(N, TBLK),)
    return pl.pallas_call(
        _transpose_body,
        grid=grid,
        in_specs=[
            pl.BlockSpec((F, TBLK), lambda i: (0, i)),
            pl.BlockSpec((F, F), lambda i: (0, 0)),
        ],
        out_specs=pl.BlockSpec((TBLK, F), lambda i: (i, 0)),
        out_shape=jax.ShapeDtypeStruct((N, F), jnp.float32),
    )(xT, eye)


def _transpose_pack4_body(xT_ref, sel_ref, out_ref):
    Q = TBLK // 4
    acc = None
    for q in range(4):
        y = lax.dot_general(
            xT_ref[:, q * Q:(q + 1) * Q], sel_ref[:, 128 * q:128 * (q + 1)],
            (((0,), (0,)), ((), ())), preferred_element_type=jnp.float32)
        acc = y if acc is None else acc + y
    out_ref[...] = acc


def _to_row_major_pack4(xT):
    F, N = xT.shape
    sel_np = np.zeros((32, 512), np.float32)
    for q in range(4):
        sel_np[np.arange(32), 160 * q + np.arange(32)] = 1.0
    sel = jnp.asarray(sel_np)
    nblk = pl.cdiv(N, TBLK)
    return pl.pallas_call(
        _transpose_pack4_body,
        grid=(nblk,),
        in_specs=[
            pl.BlockSpec((F, TBLK), lambda i: (0, i)),
            pl.BlockSpec((32, 512), lambda i: (0, 0)),
        ],
        out_specs=pl.BlockSpec((TBLK // 4, 128), lambda i: (i, 0)),
        out_shape=jax.ShapeDtypeStruct((nblk * (TBLK // 4), 128), jnp.float32),
    )(xT, sel)


def _sc_mesh():
    return plsc.VectorSubcoreMesh(
        core_axis_name="c", subcore_axis_name="s",
        num_cores=NC, num_subcores=NS)


@functools.lru_cache(maxsize=None)
def _make_sc_gather_mr():
    @functools.partial(
        pl.kernel,
        out_type=(
            jax.ShapeDtypeStruct((B, 32), jnp.float32),
            jax.ShapeDtypeStruct((B, 8), jnp.float32),
        ),
        mesh=_sc_mesh(),
        scratch_types=(
            pltpu.VMEM((NCHUNK, CH), jnp.int32),
            pltpu.VMEM((NCHUNK, CH), jnp.int32),
            pltpu.VMEM((P, 32), jnp.float32),
            pltpu.VMEM((P, 8), jnp.float32),
            pltpu.SemaphoreType.DMA,
        ),
    )
    def _sc_gather(movie_hbm, region_hbm, mt_hbm, rt_hbm,
                   m_out, r_out, midx, ridx, mrows, rrows, sem):
        wid = lax.axis_index("s") * NC + lax.axis_index("c")
        row0 = wid * NCHUNK

        pltpu.sync_copy(movie_hbm.at[pl.ds(row0, NCHUNK)], midx)
        pltpu.sync_copy(region_hbm.at[pl.ds(row0, NCHUNK)], ridx)

        base = wid * BPW
        for h in range(PH):
            goff = h * (P // 16)

            def body(g, carry):
                i16 = g * 16
                j = (goff * 16 + i16) // CH
                k = (goff * 16 + i16) % CH
                mvec = midx[j, pl.ds(k, 16)]
                rvec = ridx[j, pl.ds(k, 16)]
                for l in range(16):
                    pltpu.async_copy(mt_hbm.at[mvec[l]], mrows.at[i16 + l], sem)
                    pltpu.async_copy(rt_hbm.at[rvec[l]], rrows.at[i16 + l], sem)
                return carry

            lax.fori_loop(0, P // 16, body, 0)

            off = base + h * P
            pltpu.make_async_copy(m_out.at[pl.ds(off, P)], mrows, sem).wait()
            pltpu.make_async_copy(r_out.at[pl.ds(off, P)], rrows, sem).wait()
            pltpu.sync_copy(mrows, m_out.at[pl.ds(off, P)])
            pltpu.sync_copy(rrows, r_out.at[pl.ds(off, P)])

    return _sc_gather


@functools.lru_cache(maxsize=None)
def _make_sc_gather_u():
    @functools.partial(
        pl.kernel,
        out_type=jax.ShapeDtypeStruct((B, 128), jnp.float32),
        mesh=_sc_mesh(),
        scratch_types=(
            pltpu.VMEM((NCHUNK, CH), jnp.int32),
            pltpu.VMEM((BPW, 128), jnp.float32),
            pltpu.SemaphoreType.DMA,
        ),
    )
    def _sc_gather(user_hbm, ut_hbm, u_out, uidx, urows, sem):
        wid = lax.axis_index("s") * NC + lax.axis_index("c")
        row0 = wid * NCHUNK

        pltpu.sync_copy(user_hbm.at[pl.ds(row0, NCHUNK)], uidx)

        def body(g, carry):
            i16 = g * 16
            j = i16 // CH
            k = i16 % CH
            uvec = uidx[j, pl.ds(k, 16)]
            for l in range(16):
                pltpu.async_copy(ut_hbm.at[uvec[l]], urows.at[i16 + l], sem)
            return carry

        lax.fori_loop(0, BPW // 16, body, 0)

        base = wid * BPW
        pltpu.make_async_copy(u_out.at[pl.ds(base, BPW)], urows, sem).wait()
        pltpu.sync_copy(urows, u_out.at[pl.ds(base, BPW)])

    return _sc_gather


BLK = 2048


def _select32(w128, pos):
    out = w128[:, 0:32]
    for p in (1, 2, 3):
        out = jnp.where(pos == p, w128[:, 32 * p:32 * p + 32], out)
    return out


def _dense_body(u_ref, m_ref, r_ref, kw_ref, age_ref, upos_ref,
                kwW_ref, kwb_ref, W1u_ref, W1m_ref, W1r_ref, W1k_ref,
                w1a_ref, b1_ref, W2_ref, b2_ref, out_ref):
    u = _select32(u_ref[...], upos_ref[...])
    k = jnp.maximum(
        jnp.dot(kw_ref[...], kwW_ref[...], preferred_element_type=jnp.float32)
        + kwb_ref[...], 0.0)
    acc = jnp.dot(u, W1u_ref[...], preferred_element_type=jnp.float32)
    acc += jnp.dot(m_ref[...], W1m_ref[...], preferred_element_type=jnp.float32)
    acc += jnp.dot(r_ref[...], W1r_ref[...], preferred_element_type=jnp.float32)
    acc += jnp.dot(k, W1k_ref[...], preferred_element_type=jnp.float32)
    acc += age_ref[...] * w1a_ref[...]
    h = jnp.maximum(acc + b1_ref[...], 0.0)
    o = jnp.dot(h, W2_ref[...], preferred_element_type=jnp.float32) + b2_ref[...]
    out_ref[...] = 1.0 / (1.0 + jnp.exp(-o))


def _dense(u, m, r, keywords, age2d, upos, kwW, kwb, W1u, W1m, W1r, W1k,
           w1a, b1, W2, b2):
    grid = (B // BLK,)
    blk = lambda w: pl.BlockSpec((BLK, w), lambda i: (i, 0))
    rep = lambda s0, s1: pl.BlockSpec((s0, s1), lambda i: (0, 0))
    return pl.pallas_call(
        _dense_body,
        grid=grid,
        in_specs=[
            blk(128), blk(32), blk(8), blk(64), blk(1), blk(1),
            rep(64, 16), rep(1, 16), rep(32, 64), rep(32, 64), rep(8, 64),
            rep(16, 64), rep(1, 64), rep(1, 64), rep(64, 1), rep(1, 1),
        ],
        out_specs=blk(1),
        out_shape=jax.ShapeDtypeStruct((B, 1), jnp.float32),
    )(u, m, r, keywords, age2d, upos, kwW, kwb, W1u, W1m, W1r, W1k, w1a, b1,
      W2, b2)


def kernel(user, movie, region, keywords, age, user_table, movie_table,
           region_table, kw_W, kw_b, W1, b1, W2, b2):
    mt_rm = _to_row_major(movie_table.T)
    rt_rm = _to_row_major(region_table.T)
    m, r = _make_sc_gather_mr()(
        movie.reshape(NW * NCHUNK, CH),
        region.reshape(NW * NCHUNK, CH),
        mt_rm, rt_rm)
    ut_p4 = _to_row_major_pack4(user_table.T)
    Q = TBLK // 4
    urow = (user // TBLK) * Q + (user % Q)
    upos = (user // Q) % 4
    u = _make_sc_gather_u()(urow.reshape(NW * NCHUNK, CH), ut_p4)
    out = _dense(
        u, m, r, keywords, age[:, None], upos[:, None],
        kw_W, kw_b[None, :],
        W1[0:32], W1[32:64], W1[64:72], W1[72:88], W1[88:89],
        b1[None, :], W2, b2[None, :])
    return out[:, 0]


# speedup vs baseline: 1.4203x; 1.0127x over previous
import functools

import jax
import jax.numpy as jnp
import numpy as np
from jax import lax
from jax.experimental import pallas as pl
from jax.experimental.pallas import tpu as pltpu
from jax.experimental.pallas import tpu_sc as plsc

B = 16384
NC, NS = 2, 16
NW = NC * NS
BPW = B // NW
CH = 128
NCHUNK = BPW // CH
PH = 2
P = BPW // PH

TBLK = 32768


def _transpose_body(xT_ref, eye_ref, out_ref):
    out_ref[...] = lax.dot_general(
        xT_ref[...], eye_ref[...], (((0,), (0,)), ((), ())),
        preferred_element_type=jnp.float32)


def _to_row_major(xT):
    F, N = xT.shape
    eye = jnp.eye(F, dtype=jnp.float32)
    grid = (pl.cdiv(N, TBLK),)
    return pl.pallas_call(
        _transpose_body,
        grid=grid,
        in_specs=[
            pl.BlockSpec((F, TBLK), lambda i: (0, i)),
            pl.BlockSpec((F, F), lambda i: (0, 0)),
        ],
        out_specs=pl.BlockSpec((TBLK, F), lambda i: (i, 0)),
        out_shape=jax.ShapeDtypeStruct((N, F), jnp.float32),
    )(xT, eye)


def _transpose_pack4_body(xT_ref, sel_ref, out_ref):
    Q = TBLK // 4
    acc = None
    for q in range(4):
        y = lax.dot_general(
            xT_ref[:, q * Q:(q + 1) * Q], sel_ref[:, 128 * q:128 * (q + 1)],
            (((0,), (0,)), ((), ())), preferred_element_type=jnp.float32)
        acc = y if acc is None else acc + y
    out_ref[...] = acc


def _to_row_major_pack4(xT):
    F, N = xT.shape
    sel_np = np.zeros((32, 512), np.float32)
    for q in range(4):
        sel_np[np.arange(32), 160 * q + np.arange(32)] = 1.0
    sel = jnp.asarray(sel_np)
    nblk = pl.cdiv(N, TBLK)
    return pl.pallas_call(
        _transpose_pack4_body,
        grid=(nblk,),
        in_specs=[
            pl.BlockSpec((F, TBLK), lambda i: (0, i)),
            pl.BlockSpec((32, 512), lambda i: (0, 0)),
        ],
        out_specs=pl.BlockSpec((TBLK // 4, 128), lambda i: (i, 0)),
        out_shape=jax.ShapeDtypeStruct((nblk * (TBLK // 4), 128), jnp.float32),
    )(xT, sel)


def _sc_mesh():
    return plsc.VectorSubcoreMesh(
        core_axis_name="c", subcore_axis_name="s",
        num_cores=NC, num_subcores=NS)


@functools.lru_cache(maxsize=None)
def _make_sc_gather_mr():
    @functools.partial(
        pl.kernel,
        out_type=(
            jax.ShapeDtypeStruct((B, 32), jnp.float32),
            jax.ShapeDtypeStruct((B, 8), jnp.float32),
        ),
        mesh=_sc_mesh(),
        scratch_types=(
            pltpu.VMEM((NCHUNK, CH), jnp.int32),
            pltpu.VMEM((NCHUNK, CH), jnp.int32),
            pltpu.VMEM((P, 32), jnp.float32),
            pltpu.VMEM((P, 8), jnp.float32),
            pltpu.SemaphoreType.DMA,
        ),
    )
    def _sc_gather(movie_hbm, region_hbm, mt_hbm, rt_hbm,
                   m_out, r_out, midx, ridx, mrows, rrows, sem):
        wid = lax.axis_index("s") * NC + lax.axis_index("c")
        row0 = wid * NCHUNK

        pltpu.sync_copy(movie_hbm.at[pl.ds(row0, NCHUNK)], midx)
        pltpu.sync_copy(region_hbm.at[pl.ds(row0, NCHUNK)], ridx)

        base = wid * BPW
        for h in range(PH):
            goff = h * (P // 16)

            def body(g, carry):
                i16 = g * 16
                j = (goff * 16 + i16) // CH
                k = (goff * 16 + i16) % CH
                mvec = midx[j, pl.ds(k, 16)]
                rvec = ridx[j, pl.ds(k, 16)]
                for l in range(16):
                    pltpu.async_copy(mt_hbm.at[mvec[l]], mrows.at[i16 + l], sem)
                    pltpu.async_copy(rt_hbm.at[rvec[l]], rrows.at[i16 + l], sem)
                return carry

            lax.fori_loop(0, P // 16, body, 0)

            off = base + h * P
            pltpu.make_async_copy(m_out.at[pl.ds(off, P)], mrows, sem).wait()
            pltpu.make_async_copy(r_out.at[pl.ds(off, P)], rrows, sem).wait()
            pltpu.sync_copy(mrows, m_out.at[pl.ds(off, P)])
            pltpu.sync_copy(rrows, r_out.at[pl.ds(off, P)])

    return _sc_gather


@functools.lru_cache(maxsize=None)
def _make_sc_gather_u():
    @functools.partial(
        pl.kernel,
        out_type=jax.ShapeDtypeStruct((B, 128), jnp.float32),
        mesh=_sc_mesh(),
        scratch_types=(
            pltpu.VMEM((NCHUNK, CH), jnp.int32),
            pltpu.VMEM((BPW, 128), jnp.float32),
            pltpu.SemaphoreType.DMA,
        ),
    )
    def _sc_gather(user_hbm, ut_hbm, u_out, uidx, urows, sem):
        wid = lax.axis_index("s") * NC + lax.axis_index("c")
        row0 = wid * NCHUNK

        pltpu.sync_copy(user_hbm.at[pl.ds(row0, NCHUNK)], uidx)

        def body(g, carry):
            i16 = g * 16
            j = i16 // CH
            k = i16 % CH
            uvec = uidx[j, pl.ds(k, 16)]
            for l in range(16):
                pltpu.async_copy(ut_hbm.at[uvec[l]], urows.at[i16 + l], sem)
            return carry

        lax.fori_loop(0, BPW // 16, body, 0)

        base = wid * BPW
        pltpu.make_async_copy(u_out.at[pl.ds(base, BPW)], urows, sem).wait()
        pltpu.sync_copy(urows, u_out.at[pl.ds(base, BPW)])

    return _sc_gather


BLK = 2048


def _select32(w128, pos):
    out = w128[:, 0:32]
    for p in (1, 2, 3):
        out = jnp.where(pos == p, w128[:, 32 * p:32 * p + 32], out)
    return out


def _dense_body(u_ref, m_ref, r_ref, kwT_ref, age_ref, upos_ref,
                kwWT_ref, kwbT_ref, eye16_ref, W1u_ref, W1m_ref, W1r_ref,
                W1k_ref, w1a_ref, b1_ref, W2_ref, b2_ref, out_ref):
    u = _select32(u_ref[...], upos_ref[...])
    kT = jnp.maximum(
        jnp.dot(kwWT_ref[...], kwT_ref[...], preferred_element_type=jnp.float32)
        + kwbT_ref[...], 0.0)
    k = lax.dot_general(kT, eye16_ref[...], (((0,), (0,)), ((), ())),
                        preferred_element_type=jnp.float32)
    acc = jnp.dot(u, W1u_ref[...], preferred_element_type=jnp.float32)
    acc += jnp.dot(m_ref[...], W1m_ref[...], preferred_element_type=jnp.float32)
    acc += jnp.dot(r_ref[...], W1r_ref[...], preferred_element_type=jnp.float32)
    acc += jnp.dot(k, W1k_ref[...], preferred_element_type=jnp.float32)
    acc += age_ref[...] * w1a_ref[...]
    h = jnp.maximum(acc + b1_ref[...], 0.0)
    o = jnp.dot(h, W2_ref[...], preferred_element_type=jnp.float32) + b2_ref[...]
    out_ref[...] = 1.0 / (1.0 + jnp.exp(-o))


def _dense(u, m, r, kwT, age2d, upos, kwWT, kwbT, W1u, W1m, W1r, W1k,
           w1a, b1, W2, b2):
    grid = (B // BLK,)
    blk = lambda w: pl.BlockSpec((BLK, w), lambda i: (i, 0))
    rep = lambda s0, s1: pl.BlockSpec((s0, s1), lambda i: (0, 0))
    eye16 = jnp.asarray(np.eye(16, dtype=np.float32))
    return pl.pallas_call(
        _dense_body,
        grid=grid,
        in_specs=[
            blk(128), blk(32), blk(8), pl.BlockSpec((64, BLK), lambda i: (0, i)),
            blk(1), blk(1),
            rep(16, 64), rep(16, 1), rep(16, 16), rep(32, 64), rep(32, 64),
            rep(8, 64), rep(16, 64), rep(1, 64), rep(1, 64), rep(64, 1),
            rep(1, 1),
        ],
        out_specs=blk(1),
        out_shape=jax.ShapeDtypeStruct((B, 1), jnp.float32),
    )(u, m, r, kwT, age2d, upos, kwWT, kwbT, eye16, W1u, W1m, W1r, W1k, w1a,
      b1, W2, b2)


def kernel(user, movie, region, keywords, age, user_table, movie_table,
           region_table, kw_W, kw_b, W1, b1, W2, b2):
    mt_rm = _to_row_major(movie_table.T)
    rt_rm = _to_row_major(region_table.T)
    m, r = _make_sc_gather_mr()(
        movie.reshape(NW * NCHUNK, CH),
        region.reshape(NW * NCHUNK, CH),
        mt_rm, rt_rm)
    ut_p4 = _to_row_major_pack4(user_table.T)
    Q = TBLK // 4
    urow = (user // TBLK) * Q + (user % Q)
    upos = (user // Q) % 4
    u = _make_sc_gather_u()(urow.reshape(NW * NCHUNK, CH), ut_p4)
    out = _dense(
        u, m, r, keywords.T, age[:, None], upos[:, None],
        kw_W.T, kw_b[:, None],
        W1[0:32], W1[32:64], W1[64:72], W1[72:88], W1[88:89],
        b1[None, :], W2, b2[None, :])
    return out[:, 0]
